# baseline (speedup 1.0000x reference)
import functools

import jax
import jax.numpy as jnp
from jax.experimental import pallas as pl
from jax.experimental.pallas import tpu as pltpu
from jax.experimental.pallas import tpu_sc as plsc

N_CODES_K = 8192
DIM = 32
TILE_N = 256


def _vq_tile_body(x_ref, cb2_ref, cbsq_ref, oh_ref, idx_ref):
    x = x_ref[...]
    cb2 = cb2_ref[...]
    sq1 = jnp.sum(x * x, axis=1, keepdims=True)
    sq2 = cbsq_ref[...]
    cross2 = jax.lax.dot_general(
        x.astype(jnp.bfloat16), cb2.astype(jnp.bfloat16),
        dimension_numbers=(((1,), (1,)), ((), ())),
        preferred_element_type=jnp.float32,
    )
    dist = (sq1 + sq2) - cross2
    minval = jnp.min(dist, axis=1, keepdims=True)
    iota = jax.lax.broadcasted_iota(jnp.int32, (TILE_N, N_CODES_K), 1)
    idx = jnp.min(jnp.where(dist == minval, iota, N_CODES_K), axis=1)
    idx_ref[0, 0, :] = idx
    oh_ref[...] = (iota == idx[:, None]).astype(jnp.float32)


def _vq_onehot_and_indices(flat_x, codebook2, cb_sq):
    n = flat_x.shape[0]
    grid = n // TILE_N
    return pl.pallas_call(
        _vq_tile_body,
        grid=(grid,),
        in_specs=[
            pl.BlockSpec((TILE_N, DIM), lambda i: (i, 0)),
            pl.BlockSpec((N_CODES_K, DIM), lambda i: (0, 0)),
            pl.BlockSpec((1, N_CODES_K), lambda i: (0, 0)),
        ],
        out_specs=[
            pl.BlockSpec((TILE_N, N_CODES_K), lambda i: (i, 0)),
            pl.BlockSpec((1, 1, TILE_N), lambda i: (i, 0, 0)),
        ],
        out_shape=[
            jax.ShapeDtypeStruct((n, N_CODES_K), jnp.float32),
            jax.ShapeDtypeStruct((grid, 1, TILE_N), jnp.int32),
        ],
    )(flat_x, codebook2, cb_sq)


_SC_WINDOW = 128
_GATHER_W = 128


def _sc_gather_rows(codebook_padded, indices_2d, n):
    mesh = plsc.VectorSubcoreMesh(core_axis_name="core", subcore_axis_name="subcore")

    @functools.partial(
        pl.kernel,
        out_type=jax.ShapeDtypeStruct((n, _GATHER_W), codebook_padded.dtype),
        mesh=mesh,
    )
    def gather_kernel(cb_hbm, idx_hbm, out_hbm):
        def body(idx_vmem, out_vmem):
            pltpu.sync_copy(cb_hbm.at[idx_vmem.at[0]], out_vmem)

        pltpu.emit_pipeline(
            body,
            grid=(n // _SC_WINDOW,),
            in_specs=[pl.BlockSpec((1, _SC_WINDOW), index_map=lambda i: (0, i))],
            out_specs=[pl.BlockSpec((_SC_WINDOW, _GATHER_W), index_map=lambda i: (i, 0))],
            core_axis_name=("core", "subcore"),
            dimension_semantics=(pltpu.PARALLEL,),
        )(idx_hbm, out_hbm)

    return gather_kernel(codebook_padded, indices_2d)


def kernel(inputs, codebook):
    input_shape = inputs.shape
    flat_x = inputs.reshape(-1, DIM)
    n = flat_x.shape[0]
    cb_sq = jnp.sum(codebook * codebook, axis=1)[None, :]
    one_hot, idx_tiles = _vq_onehot_and_indices(flat_x, 2.0 * codebook, cb_sq)
    indices_2d = idx_tiles.reshape(1, n)
    cb_padded = jnp.pad(codebook, ((0, 0), (0, _GATHER_W - DIM)))
    quantized = _sc_gather_rows(cb_padded, indices_2d, n)[:, :DIM]
    return quantized.reshape(input_shape), one_hot


# speedup vs baseline: 1.1439x; 1.1439x over previous
import functools

import jax
import jax.numpy as jnp
from jax.experimental import pallas as pl
from jax.experimental.pallas import tpu as pltpu
from jax.experimental.pallas import tpu_sc as plsc

N_CODES_K = 8192
DIM = 32
TILE_N = 256


def _vq_tile_body(x_ref, cb_ref, cbsq_ref, oh_ref, idx_ref):
    x = x_ref[...]
    cb = cb_ref[...]
    sq1 = jnp.sum(x * x, axis=1, keepdims=True)
    sq2 = cbsq_ref[...]
    cross = jax.lax.dot_general(
        x.astype(jnp.bfloat16), cb.astype(jnp.bfloat16),
        dimension_numbers=(((1,), (1,)), ((), ())),
        preferred_element_type=jnp.float32,
    )
    dist = (sq1 + sq2) - 2.0 * cross
    minval = jnp.min(dist, axis=1, keepdims=True)
    iota = jax.lax.broadcasted_iota(jnp.int32, (TILE_N, N_CODES_K), 1)
    idx = jnp.min(jnp.where(dist == minval, iota, N_CODES_K), axis=1)
    idx_ref[0, 0, :] = idx
    oh_ref[...] = (iota == idx[:, None]).astype(jnp.float32)


def _vq_onehot_and_indices(flat_x, codebook, cb_sq):
    n = flat_x.shape[0]
    grid = n // TILE_N
    return pl.pallas_call(
        _vq_tile_body,
        grid=(grid,),
        in_specs=[
            pl.BlockSpec((TILE_N, DIM), lambda i: (i, 0)),
            pl.BlockSpec((N_CODES_K, DIM), lambda i: (0, 0)),
            pl.BlockSpec((1, N_CODES_K), lambda i: (0, 0)),
        ],
        out_specs=[
            pl.BlockSpec((TILE_N, N_CODES_K), lambda i: (i, 0)),
            pl.BlockSpec((1, 1, TILE_N), lambda i: (i, 0, 0)),
        ],
        out_shape=[
            jax.ShapeDtypeStruct((n, N_CODES_K), jnp.float32),
            jax.ShapeDtypeStruct((grid, 1, TILE_N), jnp.int32),
        ],
    )(flat_x, codebook, cb_sq)


_SC_WINDOW = 128
_GATHER_W = 128


def _sc_gather_rows(codebook_padded, indices_2d, n):
    mesh = plsc.VectorSubcoreMesh(core_axis_name="core", subcore_axis_name="subcore")

    @functools.partial(
        pl.kernel,
        out_type=jax.ShapeDtypeStruct((n, _GATHER_W), codebook_padded.dtype),
        mesh=mesh,
    )
    def gather_kernel(cb_hbm, idx_hbm, out_hbm):
        def body(idx_vmem, out_vmem):
            pltpu.sync_copy(cb_hbm.at[idx_vmem.at[0]], out_vmem)

        pltpu.emit_pipeline(
            body,
            grid=(n // _SC_WINDOW,),
            in_specs=[pl.BlockSpec((1, _SC_WINDOW), index_map=lambda i: (0, i))],
            out_specs=[pl.BlockSpec((_SC_WINDOW, _GATHER_W), index_map=lambda i: (i, 0))],
            core_axis_name=("core", "subcore"),
            dimension_semantics=(pltpu.PARALLEL,),
        )(idx_hbm, out_hbm)

    return gather_kernel(codebook_padded, indices_2d)


def kernel(inputs, codebook):
    input_shape = inputs.shape
    flat_x = inputs.reshape(-1, DIM)
    n = flat_x.shape[0]
    cb_sq = jnp.sum(codebook * codebook, axis=1)[None, :]
    one_hot, idx_tiles = _vq_onehot_and_indices(flat_x, codebook, cb_sq)
    indices_2d = idx_tiles.reshape(1, n)
    cb_padded = jnp.pad(codebook, ((0, 0), (0, _GATHER_W - DIM)))
    quantized = _sc_gather_rows(cb_padded, indices_2d, n)[:, :DIM]
    return quantized.reshape(input_shape), one_hot


# speedup vs baseline: 1.1620x; 1.0158x over previous
import functools

import jax
import jax.numpy as jnp
from jax.experimental import pallas as pl
from jax.experimental.pallas import tpu as pltpu
from jax.experimental.pallas import tpu_sc as plsc

N_CODES_K = 8192
DIM = 32
TILE_N = 512


def _vq_tile_body(x_ref, cb_ref, cbsq_ref, oh_ref, idx_ref):
    x = x_ref[...]
    cb = cb_ref[...]
    sq1 = jnp.sum(x * x, axis=1, keepdims=True)
    sq2 = cbsq_ref[...]
    cross = jax.lax.dot_general(
        x.astype(jnp.bfloat16), cb.astype(jnp.bfloat16),
        dimension_numbers=(((1,), (1,)), ((), ())),
        preferred_element_type=jnp.float32,
    )
    dist = (sq1 + sq2) - 2.0 * cross
    minval = jnp.min(dist, axis=1, keepdims=True)
    iota = jax.lax.broadcasted_iota(jnp.int32, (TILE_N, N_CODES_K), 1)
    idx = jnp.min(jnp.where(dist == minval, iota, N_CODES_K), axis=1)
    idx_ref[0, 0, :] = idx
    oh_ref[...] = (iota == idx[:, None]).astype(jnp.float32)


def _vq_onehot_and_indices(flat_x, codebook, cb_sq):
    n = flat_x.shape[0]
    grid = n // TILE_N
    return pl.pallas_call(
        _vq_tile_body,
        grid=(grid,),
        in_specs=[
            pl.BlockSpec((TILE_N, DIM), lambda i: (i, 0)),
            pl.BlockSpec((N_CODES_K, DIM), lambda i: (0, 0)),
            pl.BlockSpec((1, N_CODES_K), lambda i: (0, 0)),
        ],
        out_specs=[
            pl.BlockSpec((TILE_N, N_CODES_K), lambda i: (i, 0)),
            pl.BlockSpec((1, 1, TILE_N), lambda i: (i, 0, 0)),
        ],
        out_shape=[
            jax.ShapeDtypeStruct((n, N_CODES_K), jnp.float32),
            jax.ShapeDtypeStruct((grid, 1, TILE_N), jnp.int32),
        ],
    )(flat_x, codebook, cb_sq)


_SC_WINDOW = 128
_GATHER_W = 128


def _sc_gather_rows(codebook_padded, indices_2d, n):
    mesh = plsc.VectorSubcoreMesh(core_axis_name="core", subcore_axis_name="subcore")

    @functools.partial(
        pl.kernel,
        out_type=jax.ShapeDtypeStruct((n, _GATHER_W), codebook_padded.dtype),
        mesh=mesh,
    )
    def gather_kernel(cb_hbm, idx_hbm, out_hbm):
        def body(idx_vmem, out_vmem):
            pltpu.sync_copy(cb_hbm.at[idx_vmem.at[0]], out_vmem)

        pltpu.emit_pipeline(
            body,
            grid=(n // _SC_WINDOW,),
            in_specs=[pl.BlockSpec((1, _SC_WINDOW), index_map=lambda i: (0, i))],
            out_specs=[pl.BlockSpec((_SC_WINDOW, _GATHER_W), index_map=lambda i: (i, 0))],
            core_axis_name=("core", "subcore"),
            dimension_semantics=(pltpu.PARALLEL,),
        )(idx_hbm, out_hbm)

    return gather_kernel(codebook_padded, indices_2d)


def kernel(inputs, codebook):
    input_shape = inputs.shape
    flat_x = inputs.reshape(-1, DIM)
    n = flat_x.shape[0]
    cb_sq = jnp.sum(codebook * codebook, axis=1)[None, :]
    one_hot, idx_tiles = _vq_onehot_and_indices(flat_x, codebook, cb_sq)
    indices_2d = idx_tiles.reshape(1, n)
    cb_padded = jnp.pad(codebook, ((0, 0), (0, _GATHER_W - DIM)))
    quantized = _sc_gather_rows(cb_padded, indices_2d, n)[:, :DIM]
    return quantized.reshape(input_shape), one_hot
